# baseline (speedup 1.0000x reference)
import functools

import jax
import jax.numpy as jnp
from jax import lax
from jax.experimental import pallas as pl
from jax.experimental.pallas import tpu as pltpu
from jax.experimental.pallas import tpu_sc as plsc

NUM_CORES = 2
NUM_SUBCORES = 16
NUM_WORKERS = NUM_CORES * NUM_SUBCORES
BLK = 128
NBUF = 5

_LINEAR = pltpu.CompilerParams(use_tc_tiling_on_sc=False)


@functools.lru_cache(maxsize=None)
def _make_gather(total: int, v: int, d: int):
    chunks_per_w = total // BLK // NUM_WORKERS
    assert chunks_per_w % NBUF == 0
    mesh = plsc.VectorSubcoreMesh(core_axis_name="c", subcore_axis_name="s")

    @functools.partial(
        pl.kernel,
        out_type=jax.ShapeDtypeStruct((total, 2 * d), jnp.float32),
        mesh=mesh,
        compiler_params=_LINEAR,
        scratch_types=(
            [pltpu.VMEM((chunks_per_w, BLK), jnp.int32)]
            + [pltpu.VMEM((BLK, d), jnp.float32)] * NBUF
            + [pltpu.SemaphoreType.DMA] * (2 * NBUF)
        ),
    )
    def k2(t_hbm, idx_hbm, out_hbm, idx_v, *bufs_sems):
        wid = lax.axis_index("s") * NUM_CORES + lax.axis_index("c")
        base = wid * chunks_per_w
        pltpu.sync_copy(idx_hbm.at[pl.ds(base, chunks_per_w)], idx_v)
        bufs = bufs_sems[:NBUF]
        gs = bufs_sems[NBUF:2 * NBUF]
        ws = bufs_sems[2 * NBUF:]

        def _gather(j, par, fire):
            f = pltpu.async_copy if fire else pltpu.make_async_copy
            return f(t_hbm.at[idx_v.at[j]], bufs[par], gs[par])

        def _wb(j, par, fire):
            f = pltpu.async_copy if fire else pltpu.make_async_copy
            dst = pl.multiple_of((base + j) * BLK, BLK)
            return f(bufs[par], out_hbm.at[pl.ds(dst, BLK), pl.ds(0, d)],
                     ws[par])

        for par in range(NBUF - 1):
            _gather(par, par, True)

        last = chunks_per_w - 1

        def body(p, carry):
            for par in range(NBUF):
                j = NBUF * p + par
                _gather(j, par, False).wait()
                _wb(j, par, True)
                prev = (par - 1) % NBUF
                if par == 0:
                    @pl.when(p > 0)
                    def _():
                        _wb(j - 1, prev, False).wait()

                    @pl.when(j + NBUF - 1 <= last)
                    def _():
                        _gather(j + NBUF - 1, prev, True)
                else:
                    _wb(j - 1, prev, False).wait()

                    @pl.when(j + NBUF - 1 <= last)
                    def _():
                        _gather(j + NBUF - 1, prev, True)
            return carry

        lax.fori_loop(0, chunks_per_w // NBUF, body, 0)
        _wb(last, last % NBUF, False).wait()

    return k2


def kernel(stacks, table):
    batch, hist = stacks.shape
    v, d = table.shape
    total = batch * hist
    idx = stacks.reshape(total // BLK, BLK).astype(jnp.int32)
    out = _make_gather(total, v, d)(table, idx)
    return out[:, :d].reshape(batch, hist, d)
